# baseline (speedup 1.0000x reference)
import functools

import jax
import jax.numpy as jnp
from jax import lax
from jax.experimental import pallas as pl
from jax.experimental.pallas import tpu as pltpu
from jax.experimental.pallas import tpu_sc as plsc

_B, _H, _N_FROM, _C = 4, 8, 2562, 256
_TARGET = 10242
_ROWS = _B * _H * _TARGET
_XROWS = _B * _H * _N_FROM
_K = 64
_NCHUNK = _ROWS // _K
_NC, _NS = 2, 16
_NW = _NC * _NS
_RING = _NCHUNK // _NW
_LANES = 16


@functools.partial(
    pl.kernel,
    mesh=plsc.VectorSubcoreMesh(core_axis_name="c", subcore_axis_name="s"),
    out_type=jax.ShapeDtypeStruct((_ROWS, _C), jnp.float32),
    scratch_types=[
        pltpu.VMEM((2 * _K,), jnp.int32),
        pltpu.VMEM((2 * _K,), jnp.int32),
        pltpu.VMEM((2 * _K, _C), jnp.float32),
        pltpu.VMEM((2 * _K, _C), jnp.float32),
        pltpu.VMEM((_K, _C), jnp.float32),
        pltpu.VMEM((_K, _C), jnp.float32),
        pltpu.SemaphoreType.DMA,
        pltpu.SemaphoreType.DMA,
        pltpu.SemaphoreType.DMA,
        pltpu.SemaphoreType.DMA,
        pltpu.SemaphoreType.DMA,
        pltpu.SemaphoreType.DMA,
    ],
)
def _unpool(x_hbm, i01_hbm, out_hbm,
            idx_v0, idx_v1, r_v0, r_v1, o_v0, o_v1,
            sg0, sg1, so0, so1, si0, si1):
    wid = lax.axis_index("s") * _NC + lax.axis_index("c")
    idx_v = (idx_v0, idx_v1)
    r_v = (r_v0, r_v1)
    o_v = (o_v0, o_v1)
    sg = (sg0, sg1)
    so = (so0, so1)
    si = (si0, si1)

    def wait_out(slot):
        pltpu.make_async_copy(
            o_v[slot], out_hbm.at[pl.ds(0, _K)], so[slot]
        ).wait()

    def compute(slot, is_copy):
        @pl.when(is_copy)
        def _():
            def mv_body(j, cc):
                for col in range(_C // _LANES):
                    s = pl.ds(col * _LANES, _LANES)
                    o_v[slot][j, s] = r_v[slot][j, s]
                return cc

            lax.fori_loop(0, _K, mv_body, 0, unroll=4)

        @pl.when(jnp.logical_not(is_copy))
        def _():
            def row_body(j, cc):
                for col in range(_C // _LANES):
                    s = pl.ds(col * _LANES, _LANES)
                    o_v[slot][j, s] = (
                        r_v[slot][j, s] + r_v[slot][_K + j, s]
                    ) * jnp.float32(0.5)
                return cc

            lax.fori_loop(0, _K, row_body, 0, unroll=4)

    def idx_slice(c):
        return i01_hbm.at[pl.ds(c * 2 * _K, 2 * _K)]

    def chunk_is_copy(c):
        r = (c * _K) % _TARGET
        return r <= _N_FROM - _K

    def issue_idx(c, slot):
        pltpu.async_copy(idx_slice(c), idx_v[slot], si[slot])

    def wait_idx(slot):
        pltpu.make_async_copy(idx_slice(0), idx_v[slot], si[slot]).wait()

    def issue_gather(c, slot):
        is_copy = chunk_is_copy(c)

        @pl.when(is_copy)
        def _():
            pltpu.async_copy(
                x_hbm.at[idx_v[slot].at[pl.ds(0, _K)]],
                r_v[slot].at[pl.ds(0, _K)],
                sg[slot],
            )

        @pl.when(jnp.logical_not(is_copy))
        def _():
            pltpu.async_copy(x_hbm.at[idx_v[slot]], r_v[slot], sg[slot])

    def wait_fetch(c, slot):
        is_copy = chunk_is_copy(c)

        @pl.when(is_copy)
        def _():
            pltpu.make_async_copy(
                x_hbm.at[idx_v[slot].at[pl.ds(0, _K)]],
                r_v[slot].at[pl.ds(0, _K)],
                sg[slot],
            ).wait()

        @pl.when(jnp.logical_not(is_copy))
        def _():
            pltpu.make_async_copy(
                x_hbm.at[idx_v[slot]], r_v[slot], sg[slot]
            ).wait()

    pltpu.sync_copy(idx_slice(wid), idx_v0)
    issue_gather(wid, 0)
    issue_idx(_NW + wid, 1)

    def one_iter(k, p, q):
        c = k * _NW + wid

        @pl.when(k + 1 < _RING)
        def _():
            wait_idx(q)
            issue_gather((k + 1) * _NW + wid, q)

        wait_fetch(c, p)

        @pl.when(k + 2 < _RING)
        def _():
            issue_idx((k + 2) * _NW + wid, p)

        @pl.when(k >= 2)
        def _():
            wait_out(p)

        compute(p, chunk_is_copy(c))
        pltpu.async_copy(o_v[p], out_hbm.at[pl.ds(c * _K, _K)], so[p])

    def pair_body(t, carry):
        one_iter(t * 2, 0, 1)
        one_iter(t * 2 + 1, 1, 0)
        return carry

    lax.fori_loop(0, _RING // 2, pair_body, 0, unroll=False)
    wait_out(0)
    wait_out(1)

    @pl.when(wid == 0)
    def _():
        c = _RING * _NW
        pltpu.sync_copy(idx_slice(c), idx_v0)
        issue_gather(c, 0)
        wait_fetch(c, 0)
        compute(0, chunk_is_copy(c))
        pltpu.sync_copy(o_v0, out_hbm.at[pl.ds(c * _K, _K)])


def kernel(x, upsample_indices):
    up = upsample_indices.astype(jnp.int32)
    x_flat = x.transpose(0, 2, 1, 3).reshape(_XROWS, _C)
    rows = jnp.arange(_N_FROM, dtype=jnp.int32)
    pair = jnp.concatenate([jnp.stack([rows, rows], axis=1), up], axis=0)
    bh = jnp.arange(_B * _H, dtype=jnp.int32)
    off = (bh // _H) * (_N_FROM * _H) + (bh % _H)
    gid = pair[None, :, :] * _H + off[:, None, None]
    i0 = gid[:, :, 0].reshape(_NCHUNK, _K)
    i1 = gid[:, :, 1].reshape(_NCHUNK, _K)
    i01 = jnp.concatenate([i0, i1], axis=1).reshape(-1)
    out = _unpool(x_flat, i01)
    return out.reshape(_B, _H * _TARGET, _C)


# speedup vs baseline: 2.2609x; 2.2609x over previous
import functools

import jax
import jax.numpy as jnp
from jax import lax
from jax.experimental import pallas as pl
from jax.experimental.pallas import tpu as pltpu
from jax.experimental.pallas import tpu_sc as plsc

_B, _H, _N_FROM, _C = 4, 8, 2562, 256
_TARGET = 10242
_ROWS = _B * _H * _TARGET
_XROWS = _B * _H * _N_FROM
_K = 64
_NCHUNK = _ROWS // _K
_NC, _NS = 2, 16
_NW = _NC * _NS
_RING = _NCHUNK // _NW
_LANES = 16


@functools.partial(
    pl.kernel,
    mesh=plsc.VectorSubcoreMesh(core_axis_name="c", subcore_axis_name="s"),
    out_type=jax.ShapeDtypeStruct((_ROWS, _C), jnp.float32),
    scratch_types=[
        pltpu.VMEM((2 * _K,), jnp.int32),
        pltpu.VMEM((2 * _K,), jnp.int32),
        pltpu.VMEM((2 * _K, _C), jnp.float32),
        pltpu.VMEM((2 * _K, _C), jnp.float32),
        pltpu.VMEM((_K, _C), jnp.float32),
        pltpu.VMEM((_K, _C), jnp.float32),
        pltpu.SemaphoreType.DMA,
        pltpu.SemaphoreType.DMA,
        pltpu.SemaphoreType.DMA,
        pltpu.SemaphoreType.DMA,
        pltpu.SemaphoreType.DMA,
        pltpu.SemaphoreType.DMA,
    ],
)
def _unpool(x_hbm, i01_hbm, out_hbm,
            idx_v0, idx_v1, r_v0, r_v1, o_v0, o_v1,
            sg0, sg1, so0, so1, si0, si1):
    wid = lax.axis_index("s") * _NC + lax.axis_index("c")
    idx_v = (idx_v0, idx_v1)
    r_v = (r_v0, r_v1)
    o_v = (o_v0, o_v1)
    sg = (sg0, sg1)
    so = (so0, so1)
    si = (si0, si1)

    def wait_out(slot):
        pltpu.make_async_copy(
            o_v[slot], out_hbm.at[pl.ds(0, _K)], so[slot]
        ).wait()

    def compute(slot, is_copy):
        @pl.when(is_copy)
        def _():
            def mv_body(j, cc):
                for col in range(_C // _LANES):
                    s = pl.ds(col * _LANES, _LANES)
                    o_v[slot][j, s] = r_v[slot][j, s]
                return cc

            lax.fori_loop(0, _K, mv_body, 0, unroll=False)

        @pl.when(jnp.logical_not(is_copy))
        def _():
            def row_body(j, cc):
                for col in range(_C // _LANES):
                    s = pl.ds(col * _LANES, _LANES)
                    o_v[slot][j, s] = (
                        r_v[slot][j, s] + r_v[slot][_K + j, s]
                    ) * jnp.float32(0.5)
                return cc

            lax.fori_loop(0, _K, row_body, 0, unroll=False)

    def idx_slice(c):
        return i01_hbm.at[pl.ds(c * 2 * _K, 2 * _K)]

    def chunk_is_copy(c):
        r = (c * _K) % _TARGET
        return r <= _N_FROM - _K

    def issue_idx(c, slot):
        pltpu.async_copy(idx_slice(c), idx_v[slot], si[slot])

    def wait_idx(slot):
        pltpu.make_async_copy(idx_slice(0), idx_v[slot], si[slot]).wait()

    def issue_gather(c, slot):
        is_copy = chunk_is_copy(c)

        @pl.when(is_copy)
        def _():
            pltpu.async_copy(
                x_hbm.at[idx_v[slot].at[pl.ds(0, _K)]],
                r_v[slot].at[pl.ds(0, _K)],
                sg[slot],
            )

        @pl.when(jnp.logical_not(is_copy))
        def _():
            pltpu.async_copy(x_hbm.at[idx_v[slot]], r_v[slot], sg[slot])

    def wait_fetch(c, slot):
        is_copy = chunk_is_copy(c)

        @pl.when(is_copy)
        def _():
            pltpu.make_async_copy(
                x_hbm.at[idx_v[slot].at[pl.ds(0, _K)]],
                r_v[slot].at[pl.ds(0, _K)],
                sg[slot],
            ).wait()

        @pl.when(jnp.logical_not(is_copy))
        def _():
            pltpu.make_async_copy(
                x_hbm.at[idx_v[slot]], r_v[slot], sg[slot]
            ).wait()

    pltpu.sync_copy(idx_slice(wid), idx_v0)
    issue_gather(wid, 0)
    issue_idx(_NW + wid, 1)

    def one_iter(k, p, q):
        c = k * _NW + wid

        @pl.when(k + 1 < _RING)
        def _():
            wait_idx(q)
            issue_gather((k + 1) * _NW + wid, q)

        wait_fetch(c, p)

        @pl.when(k + 2 < _RING)
        def _():
            issue_idx((k + 2) * _NW + wid, p)

        @pl.when(k >= 2)
        def _():
            wait_out(p)

        compute(p, chunk_is_copy(c))
        pltpu.async_copy(o_v[p], out_hbm.at[pl.ds(c * _K, _K)], so[p])

    def pair_body(t, carry):
        one_iter(t * 2, 0, 1)
        one_iter(t * 2 + 1, 1, 0)
        return carry

    lax.fori_loop(0, _RING // 2, pair_body, 0, unroll=False)
    wait_out(0)
    wait_out(1)

    @pl.when(wid == 0)
    def _():
        c = _RING * _NW
        pltpu.sync_copy(idx_slice(c), idx_v0)
        issue_gather(c, 0)
        wait_fetch(c, 0)
        compute(0, chunk_is_copy(c))
        pltpu.sync_copy(o_v0, out_hbm.at[pl.ds(c * _K, _K)])


def kernel(x, upsample_indices):
    up = upsample_indices.astype(jnp.int32)
    x_flat = x.transpose(0, 2, 1, 3).reshape(_XROWS, _C)
    rows = jnp.arange(_N_FROM, dtype=jnp.int32)
    pair = jnp.concatenate([jnp.stack([rows, rows], axis=1), up], axis=0)
    bh = jnp.arange(_B * _H, dtype=jnp.int32)
    off = (bh // _H) * (_N_FROM * _H) + (bh % _H)
    gid = pair[None, :, :] * _H + off[:, None, None]
    i0 = gid[:, :, 0].reshape(_NCHUNK, _K)
    i1 = gid[:, :, 1].reshape(_NCHUNK, _K)
    i01 = jnp.concatenate([i0, i1], axis=1).reshape(-1)
    out = _unpool(x_flat, i01)
    return out.reshape(_B, _H * _TARGET, _C)


# speedup vs baseline: 2.2726x; 1.0052x over previous
import functools

import jax
import jax.numpy as jnp
from jax import lax
from jax.experimental import pallas as pl
from jax.experimental.pallas import tpu as pltpu
from jax.experimental.pallas import tpu_sc as plsc

_B, _H, _N_FROM, _C = 4, 8, 2562, 256
_TARGET = 10242
_ROWS = _B * _H * _TARGET
_XROWS = _B * _H * _N_FROM
_K = 64
_NCHUNK = _ROWS // _K
_NC, _NS = 2, 16
_NW = _NC * _NS
_RING = _NCHUNK // _NW
_LANES = 16


@functools.partial(
    pl.kernel,
    mesh=plsc.VectorSubcoreMesh(core_axis_name="c", subcore_axis_name="s"),
    out_type=jax.ShapeDtypeStruct((_ROWS, _C), jnp.float32),
    scratch_types=[
        pltpu.VMEM((2 * _K,), jnp.int32),
        pltpu.VMEM((2 * _K,), jnp.int32),
        pltpu.VMEM((2 * _K, _C), jnp.float32),
        pltpu.VMEM((2 * _K, _C), jnp.float32),
        pltpu.VMEM((_K, _C), jnp.float32),
        pltpu.VMEM((_K, _C), jnp.float32),
        pltpu.SemaphoreType.DMA,
        pltpu.SemaphoreType.DMA,
        pltpu.SemaphoreType.DMA,
        pltpu.SemaphoreType.DMA,
        pltpu.SemaphoreType.DMA,
        pltpu.SemaphoreType.DMA,
        pltpu.SemaphoreType.DMA,
        pltpu.SemaphoreType.DMA,
    ],
)
def _unpool(x_hbm, i01_hbm, out_hbm,
            idx_v0, idx_v1, r_v0, r_v1, o_v0, o_v1,
            sg0, sg1, so0, so1, si0, si1, sh0, sh1):
    wid = lax.axis_index("s") * _NC + lax.axis_index("c")
    idx_v = (idx_v0, idx_v1)
    r_v = (r_v0, r_v1)
    o_v = (o_v0, o_v1)
    sg = (sg0, sg1)
    so = (so0, so1)
    si = (si0, si1)
    sh = (sh0, sh1)

    def wait_out(slot):
        pltpu.make_async_copy(
            o_v[slot], out_hbm.at[pl.ds(0, _K)], so[slot]
        ).wait()

    def compute(slot, is_copy):
        @pl.when(is_copy)
        def _():
            def mv_body(j, cc):
                for col in range(_C // _LANES):
                    s = pl.ds(col * _LANES, _LANES)
                    o_v[slot][j, s] = r_v[slot][j, s]
                return cc

            lax.fori_loop(0, _K, mv_body, 0, unroll=False)

        @pl.when(jnp.logical_not(is_copy))
        def _():
            def row_body(j, cc):
                for col in range(_C // _LANES):
                    s = pl.ds(col * _LANES, _LANES)
                    o_v[slot][j, s] = (
                        r_v[slot][j, s] + r_v[slot][_K + j, s]
                    ) * jnp.float32(0.5)
                return cc

            lax.fori_loop(0, _K, row_body, 0, unroll=False)

    def idx_slice(c):
        return i01_hbm.at[pl.ds(c * 2 * _K, 2 * _K)]

    def chunk_is_copy(c):
        r = (c * _K) % _TARGET
        return r <= _N_FROM - _K

    def issue_idx(c, slot):
        pltpu.async_copy(idx_slice(c), idx_v[slot], si[slot])

    def wait_idx(slot):
        pltpu.make_async_copy(idx_slice(0), idx_v[slot], si[slot]).wait()

    def issue_gather(c, slot):
        is_copy = chunk_is_copy(c)

        @pl.when(is_copy)
        def _():
            pltpu.async_copy(
                x_hbm.at[idx_v[slot].at[pl.ds(0, _K)]],
                r_v[slot].at[pl.ds(0, _K)],
                sg[slot],
            )

        @pl.when(jnp.logical_not(is_copy))
        def _():
            pltpu.async_copy(
                x_hbm.at[idx_v[slot].at[pl.ds(0, _K)]],
                r_v[slot].at[pl.ds(0, _K)],
                sg[slot],
            )
            pltpu.async_copy(
                x_hbm.at[idx_v[slot].at[pl.ds(_K, _K)]],
                r_v[slot].at[pl.ds(_K, _K)],
                sh[slot],
            )

    def wait_fetch(c, slot):
        is_copy = chunk_is_copy(c)

        @pl.when(is_copy)
        def _():
            pltpu.make_async_copy(
                x_hbm.at[idx_v[slot].at[pl.ds(0, _K)]],
                r_v[slot].at[pl.ds(0, _K)],
                sg[slot],
            ).wait()

        @pl.when(jnp.logical_not(is_copy))
        def _():
            pltpu.make_async_copy(
                x_hbm.at[idx_v[slot].at[pl.ds(0, _K)]],
                r_v[slot].at[pl.ds(0, _K)],
                sg[slot],
            ).wait()
            pltpu.make_async_copy(
                x_hbm.at[idx_v[slot].at[pl.ds(_K, _K)]],
                r_v[slot].at[pl.ds(_K, _K)],
                sh[slot],
            ).wait()

    pltpu.sync_copy(idx_slice(wid), idx_v0)
    issue_gather(wid, 0)
    issue_idx(_NW + wid, 1)

    def one_iter(k, p, q):
        c = k * _NW + wid

        @pl.when(k + 1 < _RING)
        def _():
            wait_idx(q)
            issue_gather((k + 1) * _NW + wid, q)

        wait_fetch(c, p)

        @pl.when(k + 2 < _RING)
        def _():
            issue_idx((k + 2) * _NW + wid, p)

        @pl.when(k >= 2)
        def _():
            wait_out(p)

        compute(p, chunk_is_copy(c))
        pltpu.async_copy(o_v[p], out_hbm.at[pl.ds(c * _K, _K)], so[p])

    def pair_body(t, carry):
        one_iter(t * 2, 0, 1)
        one_iter(t * 2 + 1, 1, 0)
        return carry

    lax.fori_loop(0, _RING // 2, pair_body, 0, unroll=False)
    wait_out(0)
    wait_out(1)

    @pl.when(wid == 0)
    def _():
        c = _RING * _NW
        pltpu.sync_copy(idx_slice(c), idx_v0)
        issue_gather(c, 0)
        wait_fetch(c, 0)
        compute(0, chunk_is_copy(c))
        pltpu.sync_copy(o_v0, out_hbm.at[pl.ds(c * _K, _K)])


def kernel(x, upsample_indices):
    up = upsample_indices.astype(jnp.int32)
    x_flat = x.transpose(0, 2, 1, 3).reshape(_XROWS, _C)
    rows = jnp.arange(_N_FROM, dtype=jnp.int32)
    pair = jnp.concatenate([jnp.stack([rows, rows], axis=1), up], axis=0)
    bh = jnp.arange(_B * _H, dtype=jnp.int32)
    off = (bh // _H) * (_N_FROM * _H) + (bh % _H)
    gid = pair[None, :, :] * _H + off[:, None, None]
    i0 = gid[:, :, 0].reshape(_NCHUNK, _K)
    i1 = gid[:, :, 1].reshape(_NCHUNK, _K)
    i01 = jnp.concatenate([i0, i1], axis=1).reshape(-1)
    out = _unpool(x_flat, i01)
    return out.reshape(_B, _H * _TARGET, _C)
